# baseline (speedup 1.0000x reference)
import jax
import jax.numpy as jnp
from jax.experimental import pallas as pl
from jax.experimental.pallas import tpu as pltpu

_DIM = 2048
_N_EXPERTS = 16
_TOKENS = 16384
_CHUNK = 2048
_NBUF = 3
_NCHUNKS = _TOKENS // _CHUNK


def _tail(st, w_out_ref, i_out_ref, base):
    iota = jax.lax.broadcasted_iota(jnp.int32, st.shape, 0).astype(jnp.float32)
    m = jnp.max(st, axis=0, keepdims=True)
    e = jnp.exp(st - m)
    p = e / jnp.sum(e, axis=0, keepdims=True)
    v1 = jnp.max(p, axis=0, keepdims=True)
    i1 = jnp.min(jnp.where(p == v1, iota, float(_N_EXPERTS)),
                 axis=0, keepdims=True)
    p2 = jnp.where(iota == i1, -1.0, p)
    v2 = jnp.max(p2, axis=0, keepdims=True)
    i2 = jnp.min(jnp.where(p2 == v2, iota, float(_N_EXPERTS)),
                 axis=0, keepdims=True)
    s = v1 + v2
    w2t = jnp.concatenate([v1 / s, v2 / s], axis=0)
    i2t = jnp.concatenate([i1, i2], axis=0).astype(jnp.int32)
    w_out_ref[:, pl.ds(base, _CHUNK)] = w2t
    i_out_ref[:, pl.ds(base, _CHUNK)] = i2t


def _body(x_hbm, w_ref, b_ref, w_out_ref, i_out_ref, ring, sems):
    def copy(c):
        return pltpu.make_async_copy(
            x_hbm.at[pl.ds(c * _CHUNK, _CHUNK), :],
            ring.at[c % _NBUF], sems.at[c % _NBUF])

    for c in range(_NBUF):
        copy(c).start()
    for c in range(_NCHUNKS):
        copy(c).wait()
        st = jax.lax.dot_general(
            w_ref[...], ring[c % _NBUF],
            dimension_numbers=(((1,), (1,)), ((), ())),
            preferred_element_type=jnp.float32,
        ) + b_ref[...]
        if c + _NBUF < _NCHUNKS:
            copy(c + _NBUF).start()
        _tail(st, w_out_ref, i_out_ref, c * _CHUNK)


def kernel(x, W, b):
    b2 = b.reshape(_N_EXPERTS, 1)
    weights, indices = pl.pallas_call(
        _body,
        in_specs=[
            pl.BlockSpec(memory_space=pltpu.HBM),
            pl.BlockSpec(memory_space=pltpu.VMEM),
            pl.BlockSpec(memory_space=pltpu.VMEM),
        ],
        out_specs=[
            pl.BlockSpec(memory_space=pltpu.VMEM),
            pl.BlockSpec(memory_space=pltpu.VMEM),
        ],
        out_shape=[
            jax.ShapeDtypeStruct((2, _TOKENS), jnp.float32),
            jax.ShapeDtypeStruct((2, _TOKENS), jnp.int32),
        ],
        scratch_shapes=[
            pltpu.VMEM((_NBUF, _CHUNK, _DIM), jnp.float32),
            pltpu.SemaphoreType.DMA((_NBUF,)),
        ],
    )(x, W, b2)
    return (jnp.transpose(weights), jnp.transpose(indices))
